# baseline (speedup 1.0000x reference)
import jax
import jax.numpy as jnp
from jax import lax
from jax.experimental import pallas as pl
from jax.experimental.pallas import tpu as pltpu
from jax.experimental.pallas import tpu_sc as plsc

NC = 2
NS = 16
NW = NC * NS
CHUNK = 32
LANES = 16


def _emb_body(tok_hbm, tab_hbm, pos_hbm, out_hbm, idx_v, rows_v, pos_v,
              gsem, psem, ssem):
    wid = lax.axis_index("s") * NC + lax.axis_index("c")
    nb = tok_hbm.shape[0]
    seq = pos_hbm.shape[0]
    d = tab_hbm.shape[1]
    per_w = seq // NW
    nl = per_w // CHUNK
    nsteps = nl * nb
    wl0 = wid * per_w

    pltpu.sync_copy(tok_hbm.at[:, pl.ds(wl0, per_w)], idx_v)

    def start_gather(s):
        li = s // nb
        b = lax.rem(s, nb)
        slot = lax.rem(s, 3)
        pltpu.async_copy(
            tab_hbm.at[idx_v.at[b, pl.ds(li * CHUNK, CHUNK)]],
            rows_v.at[slot], gsem.at[slot])

    def start_pos(li):
        pltpu.async_copy(
            pos_hbm.at[pl.ds(wl0 + li * CHUNK, CHUNK)],
            pos_v.at[lax.rem(li, 2)], psem.at[lax.rem(li, 2)])

    def wait_gather(slot):
        pltpu.make_async_copy(
            tab_hbm.at[idx_v.at[0, pl.ds(0, CHUNK)]], rows_v.at[slot],
            gsem.at[slot]).wait()

    def wait_pos(pli):
        pltpu.make_async_copy(
            pos_hbm.at[pl.ds(0, CHUNK)], pos_v.at[pli],
            psem.at[pli]).wait()

    def wait_one_store(slot):
        pltpu.make_async_copy(
            rows_v.at[slot], out_hbm.at[0, pl.ds(0, CHUNK)],
            ssem.at[slot]).wait()

    start_pos(0)
    start_gather(0)

    def step(s, carry):
        li = s // nb
        b = lax.rem(s, nb)
        slot = lax.rem(s, 3)
        pli = lax.rem(li, 2)

        @pl.when(s + 1 < nsteps)
        def _():
            @pl.when(b == nb - 1)
            def _():
                start_pos(li + 1)

            @pl.when(s >= 2)
            def _():
                wait_one_store(lax.rem(s + 1, 3))

            start_gather(s + 1)

        wait_gather(slot)

        @pl.when(b == 0)
        def _():
            wait_pos(pli)

        @plsc.parallel_loop(0, CHUNK, unroll=2)
        def _row(r):
            for g in range(0, d // LANES, 8):
                a = [rows_v[slot, r, pl.ds((g + i) * LANES, LANES)]
                     for i in range(8)]
                p = [pos_v[pli, r, pl.ds((g + i) * LANES, LANES)]
                     for i in range(8)]
                for i in range(8):
                    rows_v[slot, r, pl.ds((g + i) * LANES, LANES)] = (
                        a[i] + p[i])

        l0 = wl0 + li * CHUNK
        pltpu.async_copy(rows_v.at[slot], out_hbm.at[b, pl.ds(l0, CHUNK)],
                         ssem.at[slot])
        return carry

    lax.fori_loop(0, nsteps, step, 0)
    wait_one_store((nsteps - 2) % 3)
    wait_one_store((nsteps - 1) % 3)


def kernel(tokens, token_table, pos_table):
    B, L = tokens.shape
    V, D = token_table.shape
    tok = tokens.astype(jnp.int32)
    mesh = plsc.VectorSubcoreMesh(
        core_axis_name="c", subcore_axis_name="s", num_cores=NC,
        num_subcores=NS
    )
    out = pl.kernel(
        _emb_body,
        out_type=jax.ShapeDtypeStruct((B, L, D), jnp.float32),
        mesh=mesh,
        scratch_types=[
            pltpu.VMEM((B, L // NW), jnp.int32),
            pltpu.VMEM((3, CHUNK, D), jnp.float32),
            pltpu.VMEM((2, CHUNK, D), jnp.float32),
            pltpu.SemaphoreType.DMA((3,)),
            pltpu.SemaphoreType.DMA((2,)),
            pltpu.SemaphoreType.DMA((3,)),
        ],
    )(tok, token_table, pos_table)
    return out


# speedup vs baseline: 1.0055x; 1.0055x over previous
import jax
import jax.numpy as jnp
from jax import lax
from jax.experimental import pallas as pl
from jax.experimental.pallas import tpu as pltpu
from jax.experimental.pallas import tpu_sc as plsc

NC = 2
NS = 16
NW = NC * NS
CHUNK = 32
LANES = 16


def _emb_body(tok_hbm, tab_hbm, pos_hbm, out_hbm, idx_v, rows_v, pos_v,
              gsem, psem, ssem):
    wid = lax.axis_index("s") * NC + lax.axis_index("c")
    nb = tok_hbm.shape[0]
    seq = pos_hbm.shape[0]
    d = tab_hbm.shape[1]
    per_w = seq // NW
    nl = per_w // CHUNK
    nsteps = nl * nb
    wl0 = wid * per_w

    pltpu.sync_copy(tok_hbm.at[:, pl.ds(wl0, per_w)], idx_v)

    def start_gather(s):
        li = s // nb
        b = lax.rem(s, nb)
        slot = lax.rem(s, 3)
        pltpu.async_copy(
            tab_hbm.at[idx_v.at[b, pl.ds(li * CHUNK, CHUNK)]],
            rows_v.at[slot], gsem.at[slot])

    def start_pos(li):
        pltpu.async_copy(
            pos_hbm.at[pl.ds(wl0 + li * CHUNK, CHUNK)],
            pos_v.at[lax.rem(li, 2)], psem.at[lax.rem(li, 2)])

    def wait_gather(slot):
        pltpu.make_async_copy(
            tab_hbm.at[idx_v.at[0, pl.ds(0, CHUNK)]], rows_v.at[slot],
            gsem.at[slot]).wait()

    def wait_pos(pli):
        pltpu.make_async_copy(
            pos_hbm.at[pl.ds(0, CHUNK)], pos_v.at[pli],
            psem.at[pli]).wait()

    def wait_one_store(slot):
        pltpu.make_async_copy(
            rows_v.at[slot], out_hbm.at[0, pl.ds(0, CHUNK)],
            ssem.at[slot]).wait()

    start_pos(0)
    start_gather(0)

    def step(s, carry):
        li = s // nb
        b = lax.rem(s, nb)
        slot = lax.rem(s, 3)
        pli = lax.rem(li, 2)

        @pl.when(s + 1 < nsteps)
        def _():
            @pl.when(b == nb - 1)
            def _():
                start_pos(li + 1)

            @pl.when(s >= 2)
            def _():
                wait_one_store(lax.rem(s + 1, 3))

            start_gather(s + 1)

        wait_gather(slot)

        @pl.when(b == 0)
        def _():
            wait_pos(pli)

        l0 = wl0 + li * CHUNK
        half = CHUNK // 2
        for h in range(2):
            @plsc.parallel_loop(h * half, (h + 1) * half, unroll=2)
            def _row(r):
                for g in range(0, d // LANES, 8):
                    a = [rows_v[slot, r, pl.ds((g + i) * LANES, LANES)]
                         for i in range(8)]
                    p = [pos_v[pli, r, pl.ds((g + i) * LANES, LANES)]
                         for i in range(8)]
                    for i in range(8):
                        rows_v[slot, r, pl.ds((g + i) * LANES, LANES)] = (
                            a[i] + p[i])

            pltpu.async_copy(
                rows_v.at[slot, pl.ds(h * half, half)],
                out_hbm.at[b, pl.ds(l0 + h * half, half)], ssem.at[slot])
        return carry

    lax.fori_loop(0, nsteps, step, 0)
    wait_one_store((nsteps - 2) % 3)
    wait_one_store((nsteps - 1) % 3)


def kernel(tokens, token_table, pos_table):
    B, L = tokens.shape
    V, D = token_table.shape
    tok = tokens.astype(jnp.int32)
    mesh = plsc.VectorSubcoreMesh(
        core_axis_name="c", subcore_axis_name="s", num_cores=NC,
        num_subcores=NS
    )
    out = pl.kernel(
        _emb_body,
        out_type=jax.ShapeDtypeStruct((B, L, D), jnp.float32),
        mesh=mesh,
        scratch_types=[
            pltpu.VMEM((B, L // NW), jnp.int32),
            pltpu.VMEM((3, CHUNK, D), jnp.float32),
            pltpu.VMEM((2, CHUNK, D), jnp.float32),
            pltpu.SemaphoreType.DMA((3,)),
            pltpu.SemaphoreType.DMA((2,)),
            pltpu.SemaphoreType.DMA((3,)),
        ],
    )(tok, token_table, pos_table)
    return out
